# baseline (device time: 11486 ns/iter reference)
import jax
import jax.numpy as jnp
from jax import lax
from jax.experimental import pallas as pl
from jax.experimental.pallas import tpu as pltpu

C = 4


def kernel(x):
    m, n = x.shape
    half = m // 2
    rows = half // C

    def body(x_ref, out_ref, xs_ref, xr_ref, xs_sems, xr_sems, ys_sems,
             yr_sems):
        my_x = lax.axis_index("x")
        my_y = lax.axis_index("y")
        my_z = lax.axis_index("z")
        xpeer = (1 - my_x, my_y, my_z)
        ypeer = (my_x, 1 - my_y, my_z)
        hbase = my_y * half
        obase = (1 - my_y) * half

        barrier_sem = pltpu.get_barrier_semaphore()
        for nbr in [xpeer, ypeer]:
            pl.semaphore_signal(
                barrier_sem, inc=1, device_id=nbr,
                device_id_type=pl.DeviceIdType.MESH,
            )
        xs_ref[...] = x_ref[pl.ds(hbase, half)].astype(jnp.bfloat16)
        pl.semaphore_wait(barrier_sem, 2)

        def x_rdma(c):
            sl = pl.ds(c * rows, rows)
            return pltpu.make_async_remote_copy(
                src_ref=xs_ref.at[sl],
                dst_ref=xr_ref.at[sl],
                send_sem=xs_sems.at[c],
                recv_sem=xr_sems.at[c],
                device_id=xpeer,
                device_id_type=pl.DeviceIdType.MESH,
            )

        def y_rdma(c, base):
            sl = pl.ds(base + c * rows, rows)
            return pltpu.make_async_remote_copy(
                src_ref=out_ref.at[sl],
                dst_ref=out_ref.at[sl],
                send_sem=ys_sems.at[c],
                recv_sem=yr_sems.at[c],
                device_id=ypeer,
                device_id_type=pl.DeviceIdType.MESH,
            )

        for c in range(C):
            x_rdma(c).start()

        for c in range(C):
            x_rdma(c).wait_recv()
            sl = pl.ds(c * rows, rows)
            out_ref[pl.ds(hbase + c * rows, rows)] = xs_ref[sl] + xr_ref[sl]
            y_rdma(c, hbase).start()

        for c in range(C):
            y_rdma(c, obase).wait_recv()

        for c in range(C):
            x_rdma(c).wait_send()
            y_rdma(c, hbase).wait_send()

    return pl.pallas_call(
        body,
        out_shape=jax.ShapeDtypeStruct((m, n), jnp.bfloat16),
        in_specs=[pl.BlockSpec(memory_space=pltpu.VMEM)],
        out_specs=pl.BlockSpec(memory_space=pltpu.VMEM),
        scratch_shapes=[
            pltpu.VMEM((half, n), jnp.bfloat16),
            pltpu.VMEM((half, n), jnp.bfloat16),
            pltpu.SemaphoreType.DMA((C,)),
            pltpu.SemaphoreType.DMA((C,)),
            pltpu.SemaphoreType.DMA((C,)),
            pltpu.SemaphoreType.DMA((C,)),
        ],
        compiler_params=pltpu.CompilerParams(collective_id=0),
    )(x)
